# baseline (speedup 1.0000x reference)
import functools

import jax
import jax.numpy as jnp
from jax import lax
from jax.experimental import pallas as pl
from jax.experimental.pallas import tpu as pltpu
from jax.experimental.pallas import tpu_sc as plsc

N = 10000
E = 320000
D = 128

_NC = 2
_NS = 16
_NW = _NC * _NS
_B = 128
_BPW = 80
_C = 16
_NCH = _BPW // _C
_EPAD = _NW * _BPW * _B
_NACC = N + 8
_RHI = 640
_RLO = _NACC - (_NS - 1) * _RHI


def _mm_body(h_ref, m_ref, wh_ref, wm_ref, norm_ref, h2_ref, m2_ref):
    h2 = jnp.dot(h_ref[...], wh_ref[...], preferred_element_type=jnp.float32)
    h2_ref[...] = h2 * norm_ref[...]
    m2_ref[...] = jnp.dot(m_ref[...], wm_ref[...], preferred_element_type=jnp.float32)


_mm = pl.pallas_call(
    _mm_body,
    out_shape=(
        jax.ShapeDtypeStruct((N, D), jnp.float32),
        jax.ShapeDtypeStruct((N, D), jnp.float32),
    ),
)


def _final_body(agg_ref, m2_ref, norm_ref, bh_ref, bm_ref, out_ref):
    s = (agg_ref[0] + agg_ref[1]) * norm_ref[...]
    s = s + bh_ref[...] + m2_ref[...] + bm_ref[...]
    out_ref[...] = jnp.maximum(s, 0.0)


_final = pl.pallas_call(
    _final_body,
    out_shape=jax.ShapeDtypeStruct((N, D), jnp.float32),
)


_mesh = plsc.VectorSubcoreMesh(core_axis_name="c", subcore_axis_name="s")


@functools.partial(
    pl.kernel,
    out_type=jax.ShapeDtypeStruct((_NC, N, D), jnp.float32),
    mesh=_mesh,
    scratch_types=[
        [pltpu.VMEM((_C, _B), jnp.int32) for _ in range(2)],
        [pltpu.VMEM((_C, _B), jnp.int32) for _ in range(2)],
        [pltpu.VMEM((_B, D), jnp.float32) for _ in range(2)],
        pltpu.VMEM_SHARED((_NACC, D), jnp.float32),
        pltpu.SemaphoreType.DMA,
        pltpu.SemaphoreType.DMA,
        pltpu.SemaphoreType.DMA,
    ],
)
def _sc_agg(h2_hbm, src_hbm, dst_hbm, zeros_hbm, out_hbm,
            src_v, dst_v, rows, acc_sh, semg, sems, semi):
    cid = lax.axis_index("c")
    sid = lax.axis_index("s")
    w = cid * _NS + sid

    @pl.when(sid < _NS - 1)
    def _():
        pltpu.sync_copy(zeros_hbm, acc_sh.at[pl.ds(sid * _RHI, _RHI)])

    @pl.when(sid == _NS - 1)
    def _():
        pltpu.sync_copy(zeros_hbm.at[pl.ds(0, _RLO)],
                        acc_sh.at[pl.ds(sid * _RHI, _RLO)])

    base = w * _BPW

    def _idx_load(c, sync):
        pb = c % 2
        off = base + c * _C
        if sync:
            pltpu.sync_copy(src_hbm.at[pl.ds(off, _C)], src_v[pb])
            pltpu.sync_copy(dst_hbm.at[pl.ds(off, _C)], dst_v[pb])
        else:
            pltpu.async_copy(src_hbm.at[pl.ds(off, _C)], src_v[pb], semi)
            pltpu.async_copy(dst_hbm.at[pl.ds(off, _C)], dst_v[pb], semi)

    def _wait_idx(c):
        pb = c % 2
        pltpu.make_async_copy(src_hbm.at[pl.ds(0, _C)], src_v[pb], semi).wait()
        pltpu.make_async_copy(dst_hbm.at[pl.ds(0, _C)], dst_v[pb], semi).wait()

    def _gather(j, b):
        c, p = j // _C, j % _C
        pltpu.async_copy(h2_hbm.at[src_v[c % 2].at[p]], rows[b], semg)

    def _scatter(j, b):
        c, p = j // _C, j % _C
        pltpu.async_copy(rows[b], acc_sh.at[dst_v[c % 2].at[p]], sems, add=True)

    def _wait_gather(b):
        pltpu.make_async_copy(h2_hbm.at[src_v[0].at[0]], rows[b], semg).wait()

    def _wait_scatter(b):
        pltpu.make_async_copy(rows[b], acc_sh.at[dst_v[0].at[0]], sems).wait()

    _idx_load(0, sync=True)
    plsc.subcore_barrier()

    _gather(0, 0)
    for j in range(_BPW):
        c, p, b = j // _C, j % _C, j % 2
        _wait_gather(b)
        _scatter(j, b)
        if j > 0:
            _wait_scatter(1 - b)
        if p == 0 and c < _NCH - 1:
            _idx_load(c + 1, sync=False)
        if j < _BPW - 1:
            if p == _C - 1:
                _wait_idx(c + 1)
            _gather(j + 1, 1 - b)
    _wait_scatter((_BPW - 1) % 2)
    plsc.subcore_barrier()

    @pl.when(sid < _NS - 1)
    def _():
        pltpu.sync_copy(acc_sh.at[pl.ds(sid * _RHI, _RHI)],
                        out_hbm.at[cid, pl.ds(sid * _RHI, _RHI)])

    @pl.when(sid == _NS - 1)
    def _():
        pltpu.sync_copy(acc_sh.at[pl.ds(sid * _RHI, N - (_NS - 1) * _RHI)],
                        out_hbm.at[cid, pl.ds(sid * _RHI, N - (_NS - 1) * _RHI)])


def kernel(h, m, wh, wm, bh, bm, norm, edge_index):
    h2, m2 = _mm(h, m, wh, wm, norm)
    npad = _EPAD - E
    src = jnp.concatenate([edge_index[0], jnp.zeros((npad,), jnp.int32)])
    dst = jnp.concatenate([edge_index[1], jnp.full((npad,), N, jnp.int32)])
    src2d = src.reshape(_EPAD // _B, _B)
    dst2d = dst.reshape(_EPAD // _B, _B)
    zeros = jnp.zeros((_RHI, D), dtype=jnp.float32)
    agg = _sc_agg(h2, src2d, dst2d, zeros)
    return _final(agg, m2, norm, bh.reshape(1, D), bm.reshape(1, D))


# speedup vs baseline: 1.0083x; 1.0083x over previous
import functools

import jax
import jax.numpy as jnp
from jax import lax
from jax.experimental import pallas as pl
from jax.experimental.pallas import tpu as pltpu
from jax.experimental.pallas import tpu_sc as plsc

N = 10000
E = 320000
D = 128

_NC = 2
_NS = 16
_NW = _NC * _NS
_B = 128
_BPW = 80
_C = 16
_NCH = _BPW // _C
_EPAD = _NW * _BPW * _B
_NACC = N + 8
_RHI = 640
_RLO = _NACC - (_NS - 1) * _RHI


def _mm_body(h_ref, m_ref, wh_ref, wm_ref, norm_ref, h2_ref, m2_ref):
    h2 = jnp.dot(h_ref[...], wh_ref[...], preferred_element_type=jnp.float32)
    h2_ref[...] = h2 * norm_ref[...]
    m2_ref[...] = jnp.dot(m_ref[...], wm_ref[...], preferred_element_type=jnp.float32)


_mm = pl.pallas_call(
    _mm_body,
    out_shape=(
        jax.ShapeDtypeStruct((N, D), jnp.float32),
        jax.ShapeDtypeStruct((N, D), jnp.float32),
    ),
)


def _final_body(agg_ref, m2_ref, norm_ref, bh_ref, bm_ref, out_ref):
    s = (agg_ref[0] + agg_ref[1]) * norm_ref[...]
    s = s + bh_ref[...] + m2_ref[...] + bm_ref[...]
    out_ref[...] = jnp.maximum(s, 0.0)


_final = pl.pallas_call(
    _final_body,
    out_shape=jax.ShapeDtypeStruct((N, D), jnp.float32),
)


_mesh = plsc.VectorSubcoreMesh(core_axis_name="c", subcore_axis_name="s")


@functools.partial(
    pl.kernel,
    out_type=jax.ShapeDtypeStruct((_NC, N, D), jnp.float32),
    mesh=_mesh,
    scratch_types=[
        [pltpu.VMEM((_B,), jnp.int32) for _ in range(3)],
        [pltpu.VMEM((_B,), jnp.int32) for _ in range(3)],
        [pltpu.VMEM((_B, D), jnp.float32) for _ in range(3)],
        pltpu.VMEM_SHARED((_NACC, D), jnp.float32),
        pltpu.SemaphoreType.DMA,
        pltpu.SemaphoreType.DMA,
        pltpu.SemaphoreType.DMA,
    ],
)
def _sc_agg(h2_hbm, src_hbm, dst_hbm, zeros_hbm, out_hbm,
            src_v, dst_v, rows, acc_sh, semg, sems, semi):
    cid = lax.axis_index("c")
    sid = lax.axis_index("s")
    w = cid * _NS + sid

    @pl.when(sid < _NS - 1)
    def _():
        pltpu.sync_copy(zeros_hbm, acc_sh.at[pl.ds(sid * _RHI, _RHI)])

    @pl.when(sid == _NS - 1)
    def _():
        pltpu.sync_copy(zeros_hbm.at[pl.ds(0, _RLO)],
                        acc_sh.at[pl.ds(sid * _RHI, _RLO)])

    base = w * _BPW * _B

    def _idx_load(j, k):
        off = base + j * _B
        pltpu.async_copy(src_hbm.at[pl.ds(off, _B)], src_v[k], semi)
        pltpu.async_copy(dst_hbm.at[pl.ds(off, _B)], dst_v[k], semi)

    def _wait_idx(k):
        pltpu.make_async_copy(src_hbm.at[pl.ds(0, _B)], src_v[k], semi).wait()
        pltpu.make_async_copy(dst_hbm.at[pl.ds(0, _B)], dst_v[k], semi).wait()

    def _gather(j, k, b):
        del j
        pltpu.async_copy(h2_hbm.at[src_v[k]], rows[b], semg)

    def _scatter(j, k, b):
        del j
        pltpu.async_copy(rows[b], acc_sh.at[dst_v[k]], sems, add=True)

    def _wait_gather(b):
        pltpu.make_async_copy(h2_hbm.at[src_v[0]], rows[b], semg).wait()

    def _wait_scatter(b):
        pltpu.make_async_copy(rows[b], acc_sh.at[dst_v[0]], sems).wait()

    def _step(j, dynamic_j=None):
        jj = j if dynamic_j is None else dynamic_j
        if j > 0:
            _wait_scatter((j - 1) % 3)
        _wait_gather(j % 3)
        _scatter(jj, j % 3, j % 3)
        if j + 1 < _BPW or dynamic_j is not None:
            _wait_idx((j + 1) % 3)
            _gather(jj + 1, (j + 1) % 3, (j + 1) % 3)
        if j + 2 < _BPW or dynamic_j is not None:
            _idx_load(jj + 2, (j + 2) % 3)

    _idx_load(0, 0)
    _idx_load(1, 1)
    _wait_idx(0)
    _gather(0, 0, 0)
    plsc.subcore_barrier()
    _step(0)

    def body(t, carry):
        for kk in range(3):
            _step(1 + kk, dynamic_j=1 + t * 3 + kk)
        return carry

    lax.fori_loop(0, 25, body, 0)

    for j in range(76, _BPW):
        _step(j)
    _wait_scatter((_BPW - 1) % 3)
    plsc.subcore_barrier()

    @pl.when(sid < _NS - 1)
    def _():
        pltpu.sync_copy(acc_sh.at[pl.ds(sid * _RHI, _RHI)],
                        out_hbm.at[cid, pl.ds(sid * _RHI, _RHI)])

    @pl.when(sid == _NS - 1)
    def _():
        pltpu.sync_copy(acc_sh.at[pl.ds(sid * _RHI, N - (_NS - 1) * _RHI)],
                        out_hbm.at[cid, pl.ds(sid * _RHI, N - (_NS - 1) * _RHI)])


def kernel(h, m, wh, wm, bh, bm, norm, edge_index):
    h2, m2 = _mm(h, m, wh, wm, norm)
    npad = _EPAD - E
    src = jnp.concatenate([edge_index[0], jnp.zeros((npad,), jnp.int32)])
    dst = jnp.concatenate([edge_index[1], jnp.full((npad,), N, jnp.int32)])
    zeros = jnp.zeros((_RHI, D), dtype=jnp.float32)
    agg = _sc_agg(h2, src, dst, zeros)
    return _final(agg, m2, norm, bh.reshape(1, D), bm.reshape(1, D))


# speedup vs baseline: 1.0425x; 1.0340x over previous
import functools

import jax
import jax.numpy as jnp
from jax import lax
from jax.experimental import pallas as pl
from jax.experimental.pallas import tpu as pltpu
from jax.experimental.pallas import tpu_sc as plsc

N = 10000
E = 320000
D = 128

_NC = 2
_NS = 16
_NW = _NC * _NS
_B = 128
_BPW = 80
_HB = 40
_EPAD = _NW * _BPW * _B
_NACC = N + 8
_RHI = 640
_RLO = _NACC - (_NS - 1) * _RHI


def _mm_body(h_ref, m_ref, wh_ref, wm_ref, norm_ref, h2_ref, m2_ref):
    h2 = jnp.dot(h_ref[...], wh_ref[...], preferred_element_type=jnp.float32)
    h2_ref[...] = h2 * norm_ref[...]
    m2_ref[...] = jnp.dot(m_ref[...], wm_ref[...], preferred_element_type=jnp.float32)


_mm = pl.pallas_call(
    _mm_body,
    out_shape=(
        jax.ShapeDtypeStruct((N, D), jnp.float32),
        jax.ShapeDtypeStruct((N, D), jnp.float32),
    ),
)


def _final_body(agg_ref, m2_ref, norm_ref, bh_ref, bm_ref, out_ref):
    s = (agg_ref[0] + agg_ref[1]) * norm_ref[...]
    s = s + bh_ref[...] + m2_ref[...] + bm_ref[...]
    out_ref[...] = jnp.maximum(s, 0.0)


_final = pl.pallas_call(
    _final_body,
    out_shape=jax.ShapeDtypeStruct((N, D), jnp.float32),
)


_mesh = plsc.VectorSubcoreMesh(core_axis_name="c", subcore_axis_name="s")


@functools.partial(
    pl.kernel,
    out_type=jax.ShapeDtypeStruct((_NC, N, D), jnp.float32),
    mesh=_mesh,
    scratch_types=[
        pltpu.VMEM((_HB, _B), jnp.int32),
        pltpu.VMEM((_HB, _B), jnp.int32),
        [pltpu.VMEM((_B, D), jnp.float32) for _ in range(2)],
        pltpu.VMEM_SHARED((_NACC, D), jnp.float32),
        pltpu.SemaphoreType.DMA,
    ],
)
def _sc_agg(h2_hbm, src_hbm, dst_hbm, zeros_hbm, out_hbm,
            src_v, dst_v, rows, acc_sh, semg):
    cid = lax.axis_index("c")
    sid = lax.axis_index("s")
    w = cid * _NS + sid

    @pl.when(sid < _NS - 1)
    def _():
        pltpu.sync_copy(zeros_hbm, acc_sh.at[pl.ds(sid * _RHI, _RHI)])

    @pl.when(sid == _NS - 1)
    def _():
        pltpu.sync_copy(zeros_hbm.at[pl.ds(0, _RLO)],
                        acc_sh.at[pl.ds(sid * _RHI, _RLO)])

    plsc.subcore_barrier()

    base = w * _BPW

    def _gather(p, b):
        pltpu.async_copy(h2_hbm.at[src_v.at[p]], rows[b], semg)

    def _wait_gather(b):
        pltpu.make_async_copy(h2_hbm.at[src_v.at[0]], rows[b], semg).wait()

    def _scatter(p, b):
        pltpu.sync_copy(rows[b], acc_sh.at[dst_v.at[p]], add=True)

    for h in range(_BPW // _HB):
        off = base + h * _HB
        pltpu.sync_copy(src_hbm.at[pl.ds(off, _HB)], src_v)
        pltpu.sync_copy(dst_hbm.at[pl.ds(off, _HB)], dst_v)
        _gather(0, 0)

        def body(t, carry):
            for k in range(2):
                p = 2 * t + k
                _gather(p + 1, 1 - k)
                _wait_gather(k)
                _scatter(p, k)
            return carry

        lax.fori_loop(0, (_HB - 2) // 2, body, 0)

        _gather(_HB - 1, 1)
        _wait_gather(0)
        _scatter(_HB - 2, 0)
        _wait_gather(1)
        _scatter(_HB - 1, 1)

    plsc.subcore_barrier()

    @pl.when(sid < _NS - 1)
    def _():
        pltpu.sync_copy(acc_sh.at[pl.ds(sid * _RHI, _RHI)],
                        out_hbm.at[cid, pl.ds(sid * _RHI, _RHI)])

    @pl.when(sid == _NS - 1)
    def _():
        pltpu.sync_copy(acc_sh.at[pl.ds(sid * _RHI, N - (_NS - 1) * _RHI)],
                        out_hbm.at[cid, pl.ds(sid * _RHI, N - (_NS - 1) * _RHI)])


def kernel(h, m, wh, wm, bh, bm, norm, edge_index):
    h2, m2 = _mm(h, m, wh, wm, norm)
    npad = _EPAD - E
    src = jnp.concatenate([edge_index[0], jnp.zeros((npad,), jnp.int32)])
    dst = jnp.concatenate([edge_index[1], jnp.full((npad,), N, jnp.int32)])
    src2d = src.reshape(_EPAD // _B, _B)
    dst2d = dst.reshape(_EPAD // _B, _B)
    zeros = jnp.zeros((_RHI, D), dtype=jnp.float32)
    agg = _sc_agg(h2, src2d, dst2d, zeros)
    return _final(agg, m2, norm, bh.reshape(1, D), bm.reshape(1, D))


# speedup vs baseline: 3.1622x; 3.0332x over previous
import functools

import jax
import jax.numpy as jnp
from jax import lax
from jax.experimental import pallas as pl
from jax.experimental.pallas import tpu as pltpu
from jax.experimental.pallas import tpu_sc as plsc

N = 10000
E = 320000
D = 128

_NC = 2
_NS = 16
_NW = _NC * _NS
_B = 128
_BPW = 80
_HB = 40
_EPAD = _NW * _BPW * _B
_ZROWS = 128
_RHI = 640
_RLO = N - (_NS - 1) * _RHI


def _mm_body(h_ref, m_ref, wh_ref, wm_ref, norm_ref, h2_ref, m2_ref):
    h2 = jnp.dot(h_ref[...], wh_ref[...], preferred_element_type=jnp.float32)
    h2_ref[pl.ds(0, N), :] = h2 * norm_ref[...]
    h2_ref[pl.ds(N, _ZROWS), :] = jnp.zeros((_ZROWS, D), jnp.float32)
    m2_ref[...] = jnp.dot(m_ref[...], wm_ref[...], preferred_element_type=jnp.float32)


_mm = pl.pallas_call(
    _mm_body,
    out_shape=(
        jax.ShapeDtypeStruct((N + _ZROWS, D), jnp.float32),
        jax.ShapeDtypeStruct((N, D), jnp.float32),
    ),
)


def _final_body(agg_ref, m2_ref, norm_ref, bh_ref, bm_ref, out_ref):
    s = (agg_ref[0] + agg_ref[1]) * norm_ref[...]
    s = s + bh_ref[...] + m2_ref[...] + bm_ref[...]
    out_ref[...] = jnp.maximum(s, 0.0)


_final = pl.pallas_call(
    _final_body,
    out_shape=jax.ShapeDtypeStruct((N, D), jnp.float32),
)


_mesh = plsc.VectorSubcoreMesh(core_axis_name="c", subcore_axis_name="s")


@functools.partial(
    pl.kernel,
    out_type=jax.ShapeDtypeStruct((_NC, N, D), jnp.float32),
    mesh=_mesh,
    scratch_types=[
        pltpu.VMEM((_HB, _B), jnp.int32),
        pltpu.VMEM((_HB, _B), jnp.int32),
        [pltpu.VMEM((_B, D), jnp.float32) for _ in range(2)],
        pltpu.VMEM_SHARED((N, D), jnp.float32),
        pltpu.SemaphoreType.DMA,
    ],
)
def _sc_agg(h2_hbm, src_hbm, dst_hbm, zeros_hbm, out_hbm,
            src_v, dst_v, rows, acc_sh, semg):
    cid = lax.axis_index("c")
    sid = lax.axis_index("s")
    w = cid * _NS + sid

    @pl.when(sid < _NS - 1)
    def _():
        pltpu.sync_copy(zeros_hbm, acc_sh.at[pl.ds(sid * _RHI, _RHI)])

    @pl.when(sid == _NS - 1)
    def _():
        pltpu.sync_copy(zeros_hbm.at[pl.ds(0, _RLO)],
                        acc_sh.at[pl.ds(sid * _RHI, _RLO)])

    plsc.subcore_barrier()

    base = w * _BPW

    def _gather(p, b):
        pltpu.async_copy(h2_hbm.at[src_v.at[p]], rows[b], semg)

    def _wait_gather(b):
        pltpu.make_async_copy(h2_hbm.at[src_v.at[0]], rows[b], semg).wait()

    def _scatter(p, b):
        pltpu.sync_copy(rows[b], acc_sh.at[dst_v.at[p]], add=True)

    for h in range(_BPW // _HB):
        off = base + h * _HB
        pltpu.sync_copy(src_hbm.at[pl.ds(off, _HB)], src_v)
        pltpu.sync_copy(dst_hbm.at[pl.ds(off, _HB)], dst_v)
        _gather(0, 0)

        def body(t, carry):
            for k in range(2):
                p = 2 * t + k
                _gather(p + 1, 1 - k)
                _wait_gather(k)
                _scatter(p, k)
            return carry

        lax.fori_loop(0, (_HB - 2) // 2, body, 0)

        _gather(_HB - 1, 1)
        _wait_gather(0)
        _scatter(_HB - 2, 0)
        _wait_gather(1)
        _scatter(_HB - 1, 1)

    plsc.subcore_barrier()

    @pl.when(sid < _NS - 1)
    def _():
        pltpu.sync_copy(acc_sh.at[pl.ds(sid * _RHI, _RHI)],
                        out_hbm.at[cid, pl.ds(sid * _RHI, _RHI)])

    @pl.when(sid == _NS - 1)
    def _():
        pltpu.sync_copy(acc_sh.at[pl.ds(sid * _RHI, _RLO)],
                        out_hbm.at[cid, pl.ds(sid * _RHI, _RLO)])


def kernel(h, m, wh, wm, bh, bm, norm, edge_index):
    h2, m2 = _mm(h, m, wh, wm, norm)
    npad = _EPAD - E
    pad_iota = jnp.arange(npad, dtype=jnp.int32)
    src = jnp.concatenate([edge_index[0], N + pad_iota % _ZROWS])
    dst = jnp.concatenate([edge_index[1], pad_iota % N])
    src2d = src.reshape(_EPAD // _B, _B)
    dst2d = dst.reshape(_EPAD // _B, _B)
    zeros = jnp.zeros((_RHI, D), dtype=jnp.float32)
    agg = _sc_agg(h2, src2d, dst2d, zeros)
    return _final(agg, m2, norm, bh.reshape(1, D), bm.reshape(1, D))
